# baseline (speedup 1.0000x reference)
import functools

import jax
import jax.numpy as jnp
from jax import lax
from jax.experimental import pallas as pl
from jax.experimental.pallas import tpu as pltpu
from jax.experimental.pallas import tpu_sc as plsc

N = 10000
E = 320000
IN_CH = 128
HID = 256
HALF = 128
NCLS = 10
NLAYERS = 3
NGRAPHS = 64

NC = 2
NS = 16
K = 128
CHUNKS = 160
G = 40
GROUPS = CHUNKS // G
E_PAD = NS * CHUNKS * K
ZR = 632
AGG_ROWS = NS * ZR
TAIL = N - 15 * ZR

RB = 1000
GRID = N // RB


def _make_edge_agg(chunks, edge_split):
    groups = chunks // G

    def body(tab_hbm, src_hbm, dst_hbm, zeros_hbm, out_hbm,
             src_v, dst_v, rows0, rows1, agg_sh, sem0, sem1):
        c = lax.axis_index("c")
        s = lax.axis_index("s")
        pltpu.sync_copy(zeros_hbm.at[pl.ds(s * ZR, ZR)],
                        agg_sh.at[pl.ds(s * ZR, ZR)])
        plsc.subcore_barrier()

        def gather(j, buf, sem):
            pltpu.async_copy(tab_hbm.at[src_v.at[j]], buf, sem)

        def wait_rows(buf, sem):
            pltpu.make_async_copy(tab_hbm.at[pl.ds(0, K)], buf, sem).wait()

        def group(g, carry):
            pltpu.sync_copy(src_hbm.at[c, s, pl.ds(g * G, G)], src_v)
            if edge_split:
                pltpu.sync_copy(dst_hbm.at[c, s, pl.ds(g * G, G)], dst_v)
            else:
                pltpu.sync_copy(dst_hbm.at[s, pl.ds(g * G, G)], dst_v)
            gather(0, rows0, sem0)

            def pair(t, carry2):
                j0 = 2 * t
                wait_rows(rows0, sem0)
                gather(j0 + 1, rows1, sem1)
                pltpu.sync_copy(rows0, agg_sh.at[dst_v.at[j0]], add=True)
                wait_rows(rows1, sem1)

                @pl.when(t < G // 2 - 1)
                def _():
                    gather(j0 + 2, rows0, sem0)

                pltpu.sync_copy(rows1, agg_sh.at[dst_v.at[j0 + 1]], add=True)
                return carry2

            return lax.fori_loop(0, G // 2, pair, carry)

        lax.fori_loop(0, groups, group, 0)
        plsc.subcore_barrier()

        @pl.when(s < NS - 1)
        def _():
            pltpu.sync_copy(agg_sh.at[pl.ds(s * ZR, ZR)],
                            out_hbm.at[pl.ds(c * N + s * ZR, ZR)])

        @pl.when(s == NS - 1)
        def _():
            pltpu.sync_copy(agg_sh.at[pl.ds((NS - 1) * ZR, TAIL)],
                            out_hbm.at[pl.ds(c * N + (NS - 1) * ZR, TAIL)])

    return pl.kernel(
        body,
        out_type=jax.ShapeDtypeStruct((NC * N, HALF), jnp.float32),
        mesh=plsc.VectorSubcoreMesh(core_axis_name="c", subcore_axis_name="s"),
        scratch_types=[
            pltpu.VMEM((G, K), jnp.int32),
            pltpu.VMEM((G, K), jnp.int32),
            pltpu.VMEM((K, HALF), jnp.float32),
            pltpu.VMEM((K, HALF), jnp.float32),
            pltpu.VMEM_SHARED((AGG_ROWS, HALF), jnp.float32),
            pltpu.SemaphoreType.DMA,
            pltpu.SemaphoreType.DMA,
        ],
    )


CHUNKS_ES = 80
_edge_agg_l1 = _make_edge_agg(CHUNKS_ES, True)
_edge_agg128 = _make_edge_agg(CHUNKS, False)


def _gru_tail(m_agg, gh, h_full, o_ref):
    gi = m_agg
    r = jax.nn.sigmoid(gi[:, :HID] + gh[:, :HID])
    z = jax.nn.sigmoid(gi[:, HID:2 * HID] + gh[:, HID:2 * HID])
    n = jnp.tanh(gi[:, 2 * HID:] + r * gh[:, 2 * HID:])
    h_new = (1.0 - z) * n + z * h_full
    o_ref[0] = h_new[:, :HALF]
    o_ref[1] = h_new[:, HALF:]


def _gru1_body(s_ref, x_ref, w_ref, wi_ref, wh_ref, bi_ref, bh_ref, o_ref):
    s = s_ref[0] + s_ref[1]
    m_agg = jnp.dot(s, w_ref[...], preferred_element_type=jnp.float32)
    gi = jnp.dot(m_agg, wi_ref[...], preferred_element_type=jnp.float32) + bi_ref[...]
    gh = jnp.dot(x_ref[...], wh_ref[...], preferred_element_type=jnp.float32) + bh_ref[...]
    h_full = jnp.concatenate(
        [x_ref[...], jnp.zeros((RB, HALF), jnp.float32)], axis=1)
    _gru_tail(gi, gh, h_full, o_ref)


def _gru1(s2, x, w1, wiT, whT1, bi, bh):
    return pl.pallas_call(
        _gru1_body,
        grid=(GRID,),
        in_specs=[pl.BlockSpec((NC, RB, HALF), lambda i: (0, i, 0)),
                  pl.BlockSpec((RB, HALF), lambda i: (i, 0)),
                  pl.BlockSpec((HALF, HID), lambda i: (0, 0)),
                  pl.BlockSpec((HID, 3 * HID), lambda i: (0, 0)),
                  pl.BlockSpec((HALF, 3 * HID), lambda i: (0, 0)),
                  pl.BlockSpec((1, 3 * HID), lambda i: (0, 0)),
                  pl.BlockSpec((1, 3 * HID), lambda i: (0, 0))],
        out_specs=pl.BlockSpec((NC, RB, HALF), lambda i: (0, i, 0)),
        out_shape=jax.ShapeDtypeStruct((NC, N, HALF), jnp.float32),
    )(s2, x, w1, wiT, whT1, bi, bh)


def _gru23_body(s_ref, h_ref, w_ref, wi_ref, wh_ref, bi_ref, bh_ref, o_ref):
    m_agg = (jnp.dot(s_ref[0], w_ref[:HALF, :], preferred_element_type=jnp.float32)
             + jnp.dot(s_ref[1], w_ref[HALF:, :], preferred_element_type=jnp.float32))
    gi = jnp.dot(m_agg, wi_ref[...], preferred_element_type=jnp.float32) + bi_ref[...]
    gh = (jnp.dot(h_ref[0], wh_ref[:HALF, :], preferred_element_type=jnp.float32)
          + jnp.dot(h_ref[1], wh_ref[HALF:, :], preferred_element_type=jnp.float32)
          + bh_ref[...])
    h_full = jnp.concatenate([h_ref[0], h_ref[1]], axis=1)
    _gru_tail(gi, gh, h_full, o_ref)


def _gru23(s2, h2, w, wiT, whT, bi, bh):
    return pl.pallas_call(
        _gru23_body,
        grid=(GRID,),
        in_specs=[pl.BlockSpec((NC, RB, HALF), lambda i: (0, i, 0)),
                  pl.BlockSpec((NC, RB, HALF), lambda i: (0, i, 0)),
                  pl.BlockSpec((HID, HID), lambda i: (0, 0)),
                  pl.BlockSpec((HID, 3 * HID), lambda i: (0, 0)),
                  pl.BlockSpec((HID, 3 * HID), lambda i: (0, 0)),
                  pl.BlockSpec((1, 3 * HID), lambda i: (0, 0)),
                  pl.BlockSpec((1, 3 * HID), lambda i: (0, 0))],
        out_specs=pl.BlockSpec((NC, RB, HALF), lambda i: (0, i, 0)),
        out_shape=jax.ShapeDtypeStruct((NC, N, HALF), jnp.float32),
    )(s2, h2, w, wiT, whT, bi, bh)


def _pool_body(h_ref, b_ref, w_ref, lb_ref, o_ref, sums, cnt):
    i = pl.program_id(0)

    @pl.when(i == 0)
    def _():
        sums[...] = jnp.zeros_like(sums)
        cnt[...] = jnp.zeros_like(cnt)

    gid = lax.broadcasted_iota(jnp.int32, (NGRAPHS, RB), 0)
    oh = (b_ref[0] == gid).astype(jnp.float32)
    hcat = jnp.concatenate([h_ref[0], h_ref[1]], axis=1)
    sums[...] += jnp.dot(oh, hcat, preferred_element_type=jnp.float32)
    cnt[...] += jnp.broadcast_to(jnp.sum(oh, axis=1, keepdims=True),
                                 (NGRAPHS, HALF))

    @pl.when(i == GRID - 1)
    def _():
        pooled = sums[...] / jnp.maximum(cnt[:, 0:1], 1.0)
        o_ref[...] = (jnp.dot(pooled, w_ref[...],
                              preferred_element_type=jnp.float32)
                      + lb_ref[...])


def _pool(h2, batch2, lin_WT, lin_b2):
    return pl.pallas_call(
        _pool_body,
        grid=(GRID,),
        in_specs=[pl.BlockSpec((NC, RB, HALF), lambda i: (0, i, 0)),
                  pl.BlockSpec((1, 1, RB), lambda i: (i, 0, 0)),
                  pl.BlockSpec((HID, NCLS), lambda i: (0, 0)),
                  pl.BlockSpec((1, NCLS), lambda i: (0, 0))],
        out_specs=pl.BlockSpec((NGRAPHS, NCLS), lambda i: (0, 0)),
        out_shape=jax.ShapeDtypeStruct((NGRAPHS, NCLS), jnp.float32),
        scratch_shapes=[pltpu.VMEM((NGRAPHS, HID), jnp.float32),
                        pltpu.VMEM((NGRAPHS, HALF), jnp.float32)],
    )(h2, batch2, lin_WT, lin_b2)


def kernel(x, edge_index, batch, weight, W_ih, W_hh, b_ih, b_hh, lin_W, lin_b):
    src = edge_index[0].astype(jnp.int32)
    dst = edge_index[1].astype(jnp.int32)
    batch = batch.astype(jnp.int32)

    pad = E_PAD - E
    srcp = jnp.concatenate([src, jnp.zeros((pad,), jnp.int32)])
    trash = N + jnp.arange(pad, dtype=jnp.int32) % (AGG_ROWS - N)
    dstp = jnp.concatenate([dst, trash])
    src_st = jnp.stack([srcp, srcp + N]).reshape(NC, NS, CHUNKS, K)
    dst3 = dstp.reshape(NS, CHUNKS, K)
    src_es = srcp.reshape(NC, NS, CHUNKS_ES, K)
    dst_es = dstp.reshape(NC, NS, CHUNKS_ES, K)
    zeros128 = jnp.zeros((AGG_ROWS, HALF), jnp.float32)

    wiT = W_ih.T
    whT = W_hh.T
    bi = b_ih.reshape(1, 3 * HID)
    bh = b_hh.reshape(1, 3 * HID)

    s1 = _edge_agg_l1(x, src_es, dst_es, zeros128)
    h2 = _gru1(s1.reshape(NC, N, HALF), x, weight[0][:HALF],
               wiT, whT[:HALF], bi, bh)

    for i in range(1, NLAYERS):
        s = _edge_agg128(h2.reshape(NC * N, HALF), src_st, dst3, zeros128)
        h2 = _gru23(s.reshape(NC, N, HALF), h2, weight[i], wiT, whT, bi, bh)

    batch2 = batch.reshape(GRID, 1, RB)
    return _pool(h2, batch2, lin_W.T, lin_b.reshape(1, NCLS))


# speedup vs baseline: 1.1083x; 1.1083x over previous
import functools

import jax
import jax.numpy as jnp
from jax import lax
from jax.experimental import pallas as pl
from jax.experimental.pallas import tpu as pltpu
from jax.experimental.pallas import tpu_sc as plsc

N = 10000
E = 320000
IN_CH = 128
HID = 256
HALF = 128
NCLS = 10
NLAYERS = 3
NGRAPHS = 64

NC = 2
NS = 16
K = 64
NBUF = 4
CHUNKS = 320
G = 40
GROUPS = CHUNKS // G
E_PAD = NS * CHUNKS * K
ZR = 632
AGG_ROWS = NS * ZR
TAIL = N - 15 * ZR

RB = 1000
GRID = N // RB


def _make_edge_agg(chunks, edge_split):
    groups = chunks // G

    def body(tab_hbm, src_hbm, dst_hbm, zeros_hbm, out_hbm,
             src_v, dst_v, rows0, rows1, rows2, rows3, agg_sh,
             sem0, sem1, sem2, sem3):
        bufs = (rows0, rows1, rows2, rows3)
        sems = (sem0, sem1, sem2, sem3)
        c = lax.axis_index("c")
        s = lax.axis_index("s")
        pltpu.sync_copy(zeros_hbm.at[pl.ds(s * ZR, ZR)],
                        agg_sh.at[pl.ds(s * ZR, ZR)])
        plsc.subcore_barrier()

        def gather(j, buf, sem):
            pltpu.async_copy(tab_hbm.at[src_v.at[j]], buf, sem)

        def wait_rows(buf, sem):
            pltpu.make_async_copy(tab_hbm.at[pl.ds(0, K)], buf, sem).wait()

        def group(g, carry):
            pltpu.sync_copy(src_hbm.at[c, s, pl.ds(g * G, G)], src_v)
            if edge_split:
                pltpu.sync_copy(dst_hbm.at[c, s, pl.ds(g * G, G)], dst_v)
            else:
                pltpu.sync_copy(dst_hbm.at[s, pl.ds(g * G, G)], dst_v)
            for b in range(NBUF - 1):
                gather(b, bufs[b], sems[b])

            def quad(t, carry2):
                j = NBUF * t
                for b in range(NBUF):
                    wait_rows(bufs[b], sems[b])

                    @pl.when(j + b + NBUF - 1 < G)
                    def _(b=b):
                        gather(j + b + NBUF - 1,
                               bufs[(b + NBUF - 1) % NBUF],
                               sems[(b + NBUF - 1) % NBUF])

                    pltpu.sync_copy(bufs[b], agg_sh.at[dst_v.at[j + b]],
                                    add=True)
                return carry2

            return lax.fori_loop(0, G // NBUF, quad, carry)

        lax.fori_loop(0, groups, group, 0)
        plsc.subcore_barrier()

        @pl.when(s < NS - 1)
        def _():
            pltpu.sync_copy(agg_sh.at[pl.ds(s * ZR, ZR)],
                            out_hbm.at[pl.ds(c * N + s * ZR, ZR)])

        @pl.when(s == NS - 1)
        def _():
            pltpu.sync_copy(agg_sh.at[pl.ds((NS - 1) * ZR, TAIL)],
                            out_hbm.at[pl.ds(c * N + (NS - 1) * ZR, TAIL)])

    return pl.kernel(
        body,
        out_type=jax.ShapeDtypeStruct((NC * N, HALF), jnp.float32),
        mesh=plsc.VectorSubcoreMesh(core_axis_name="c", subcore_axis_name="s"),
        scratch_types=[
            pltpu.VMEM((G, K), jnp.int32),
            pltpu.VMEM((G, K), jnp.int32),
            pltpu.VMEM((K, HALF), jnp.float32),
            pltpu.VMEM((K, HALF), jnp.float32),
            pltpu.VMEM((K, HALF), jnp.float32),
            pltpu.VMEM((K, HALF), jnp.float32),
            pltpu.VMEM_SHARED((AGG_ROWS, HALF), jnp.float32),
            pltpu.SemaphoreType.DMA,
            pltpu.SemaphoreType.DMA,
            pltpu.SemaphoreType.DMA,
            pltpu.SemaphoreType.DMA,
        ],
    )


CHUNKS_ES = 160
_edge_agg_l1 = _make_edge_agg(CHUNKS_ES, True)
_edge_agg128 = _make_edge_agg(CHUNKS, False)


def _gru_tail(m_agg, gh, h_full, o_ref):
    gi = m_agg
    r = jax.nn.sigmoid(gi[:, :HID] + gh[:, :HID])
    z = jax.nn.sigmoid(gi[:, HID:2 * HID] + gh[:, HID:2 * HID])
    n = jnp.tanh(gi[:, 2 * HID:] + r * gh[:, 2 * HID:])
    h_new = (1.0 - z) * n + z * h_full
    o_ref[0] = h_new[:, :HALF]
    o_ref[1] = h_new[:, HALF:]


def _gru1_body(s_ref, x_ref, w_ref, wi_ref, wh_ref, bi_ref, bh_ref, o_ref):
    s = s_ref[0] + s_ref[1]
    m_agg = jnp.dot(s, w_ref[...], preferred_element_type=jnp.float32)
    gi = jnp.dot(m_agg, wi_ref[...], preferred_element_type=jnp.float32) + bi_ref[...]
    gh = jnp.dot(x_ref[...], wh_ref[...], preferred_element_type=jnp.float32) + bh_ref[...]
    h_full = jnp.concatenate(
        [x_ref[...], jnp.zeros((RB, HALF), jnp.float32)], axis=1)
    _gru_tail(gi, gh, h_full, o_ref)


def _gru1(s2, x, w1, wiT, whT1, bi, bh):
    return pl.pallas_call(
        _gru1_body,
        grid=(GRID,),
        in_specs=[pl.BlockSpec((NC, RB, HALF), lambda i: (0, i, 0)),
                  pl.BlockSpec((RB, HALF), lambda i: (i, 0)),
                  pl.BlockSpec((HALF, HID), lambda i: (0, 0)),
                  pl.BlockSpec((HID, 3 * HID), lambda i: (0, 0)),
                  pl.BlockSpec((HALF, 3 * HID), lambda i: (0, 0)),
                  pl.BlockSpec((1, 3 * HID), lambda i: (0, 0)),
                  pl.BlockSpec((1, 3 * HID), lambda i: (0, 0))],
        out_specs=pl.BlockSpec((NC, RB, HALF), lambda i: (0, i, 0)),
        out_shape=jax.ShapeDtypeStruct((NC, N, HALF), jnp.float32),
    )(s2, x, w1, wiT, whT1, bi, bh)


def _gru23_body(s_ref, h_ref, w_ref, wi_ref, wh_ref, bi_ref, bh_ref, o_ref):
    m_agg = (jnp.dot(s_ref[0], w_ref[:HALF, :], preferred_element_type=jnp.float32)
             + jnp.dot(s_ref[1], w_ref[HALF:, :], preferred_element_type=jnp.float32))
    gi = jnp.dot(m_agg, wi_ref[...], preferred_element_type=jnp.float32) + bi_ref[...]
    gh = (jnp.dot(h_ref[0], wh_ref[:HALF, :], preferred_element_type=jnp.float32)
          + jnp.dot(h_ref[1], wh_ref[HALF:, :], preferred_element_type=jnp.float32)
          + bh_ref[...])
    h_full = jnp.concatenate([h_ref[0], h_ref[1]], axis=1)
    _gru_tail(gi, gh, h_full, o_ref)


def _gru23(s2, h2, w, wiT, whT, bi, bh):
    return pl.pallas_call(
        _gru23_body,
        grid=(GRID,),
        in_specs=[pl.BlockSpec((NC, RB, HALF), lambda i: (0, i, 0)),
                  pl.BlockSpec((NC, RB, HALF), lambda i: (0, i, 0)),
                  pl.BlockSpec((HID, HID), lambda i: (0, 0)),
                  pl.BlockSpec((HID, 3 * HID), lambda i: (0, 0)),
                  pl.BlockSpec((HID, 3 * HID), lambda i: (0, 0)),
                  pl.BlockSpec((1, 3 * HID), lambda i: (0, 0)),
                  pl.BlockSpec((1, 3 * HID), lambda i: (0, 0))],
        out_specs=pl.BlockSpec((NC, RB, HALF), lambda i: (0, i, 0)),
        out_shape=jax.ShapeDtypeStruct((NC, N, HALF), jnp.float32),
    )(s2, h2, w, wiT, whT, bi, bh)


def _pool_body(h_ref, b_ref, w_ref, lb_ref, o_ref, sums, cnt):
    i = pl.program_id(0)

    @pl.when(i == 0)
    def _():
        sums[...] = jnp.zeros_like(sums)
        cnt[...] = jnp.zeros_like(cnt)

    gid = lax.broadcasted_iota(jnp.int32, (NGRAPHS, RB), 0)
    oh = (b_ref[0] == gid).astype(jnp.float32)
    hcat = jnp.concatenate([h_ref[0], h_ref[1]], axis=1)
    sums[...] += jnp.dot(oh, hcat, preferred_element_type=jnp.float32)
    cnt[...] += jnp.broadcast_to(jnp.sum(oh, axis=1, keepdims=True),
                                 (NGRAPHS, HALF))

    @pl.when(i == GRID - 1)
    def _():
        pooled = sums[...] / jnp.maximum(cnt[:, 0:1], 1.0)
        o_ref[...] = (jnp.dot(pooled, w_ref[...],
                              preferred_element_type=jnp.float32)
                      + lb_ref[...])


def _pool(h2, batch2, lin_WT, lin_b2):
    return pl.pallas_call(
        _pool_body,
        grid=(GRID,),
        in_specs=[pl.BlockSpec((NC, RB, HALF), lambda i: (0, i, 0)),
                  pl.BlockSpec((1, 1, RB), lambda i: (i, 0, 0)),
                  pl.BlockSpec((HID, NCLS), lambda i: (0, 0)),
                  pl.BlockSpec((1, NCLS), lambda i: (0, 0))],
        out_specs=pl.BlockSpec((NGRAPHS, NCLS), lambda i: (0, 0)),
        out_shape=jax.ShapeDtypeStruct((NGRAPHS, NCLS), jnp.float32),
        scratch_shapes=[pltpu.VMEM((NGRAPHS, HID), jnp.float32),
                        pltpu.VMEM((NGRAPHS, HALF), jnp.float32)],
    )(h2, batch2, lin_WT, lin_b2)


def kernel(x, edge_index, batch, weight, W_ih, W_hh, b_ih, b_hh, lin_W, lin_b):
    src = edge_index[0].astype(jnp.int32)
    dst = edge_index[1].astype(jnp.int32)
    batch = batch.astype(jnp.int32)

    pad = E_PAD - E
    srcp = jnp.concatenate([src, jnp.zeros((pad,), jnp.int32)])
    trash = N + jnp.arange(pad, dtype=jnp.int32) % (AGG_ROWS - N)
    dstp = jnp.concatenate([dst, trash])
    src_st = jnp.stack([srcp, srcp + N]).reshape(NC, NS, CHUNKS, K)
    dst3 = dstp.reshape(NS, CHUNKS, K)
    src_es = srcp.reshape(NC, NS, CHUNKS_ES, K)
    dst_es = dstp.reshape(NC, NS, CHUNKS_ES, K)
    zeros128 = jnp.zeros((AGG_ROWS, HALF), jnp.float32)

    wiT = W_ih.T
    whT = W_hh.T
    bi = b_ih.reshape(1, 3 * HID)
    bh = b_hh.reshape(1, 3 * HID)

    s1 = _edge_agg_l1(x, src_es, dst_es, zeros128)
    h2 = _gru1(s1.reshape(NC, N, HALF), x, weight[0][:HALF],
               wiT, whT[:HALF], bi, bh)

    for i in range(1, NLAYERS):
        s = _edge_agg128(h2.reshape(NC * N, HALF), src_st, dst3, zeros128)
        h2 = _gru23(s.reshape(NC, N, HALF), h2, weight[i], wiT, whT, bi, bh)

    batch2 = batch.reshape(GRID, 1, RB)
    return _pool(h2, batch2, lin_W.T, lin_b.reshape(1, NCLS))


# speedup vs baseline: 1.1425x; 1.0308x over previous
import functools

import jax
import jax.numpy as jnp
from jax import lax
from jax.experimental import pallas as pl
from jax.experimental.pallas import tpu as pltpu
from jax.experimental.pallas import tpu_sc as plsc

N = 10000
E = 320000
IN_CH = 128
HID = 256
HALF = 128
NCLS = 10
NLAYERS = 3
NGRAPHS = 64

NC = 2
NS = 16
K = 32
NBUF = 8
CHUNKS = 640
G = 40
GROUPS = CHUNKS // G
E_PAD = NS * CHUNKS * K
ZR = 632
AGG_ROWS = NS * ZR
TAIL = N - 15 * ZR

RB = 1000
GRID = N // RB


def _make_edge_agg(chunks, edge_split):
    groups = chunks // G

    def body(tab_hbm, src_hbm, dst_hbm, zeros_hbm, out_hbm,
             src_v, dst_v, rows0, rows1, rows2, rows3, rows4, rows5,
             rows6, rows7, agg_sh,
             sem0, sem1, sem2, sem3, sem4, sem5, sem6, sem7):
        bufs = (rows0, rows1, rows2, rows3, rows4, rows5, rows6, rows7)
        sems = (sem0, sem1, sem2, sem3, sem4, sem5, sem6, sem7)
        c = lax.axis_index("c")
        s = lax.axis_index("s")
        pltpu.sync_copy(zeros_hbm.at[pl.ds(s * ZR, ZR)],
                        agg_sh.at[pl.ds(s * ZR, ZR)])
        plsc.subcore_barrier()

        def gather(j, buf, sem):
            pltpu.async_copy(tab_hbm.at[src_v.at[j]], buf, sem)

        def wait_rows(buf, sem):
            pltpu.make_async_copy(tab_hbm.at[pl.ds(0, K)], buf, sem).wait()

        def group(g, carry):
            pltpu.sync_copy(src_hbm.at[c, s, pl.ds(g * G, G)], src_v)
            if edge_split:
                pltpu.sync_copy(dst_hbm.at[c, s, pl.ds(g * G, G)], dst_v)
            else:
                pltpu.sync_copy(dst_hbm.at[s, pl.ds(g * G, G)], dst_v)
            for b in range(NBUF - 1):
                gather(b, bufs[b], sems[b])

            def quad(t, carry2):
                j = NBUF * t
                for b in range(NBUF):
                    wait_rows(bufs[b], sems[b])

                    @pl.when(j + b + NBUF - 1 < G)
                    def _(b=b):
                        gather(j + b + NBUF - 1,
                               bufs[(b + NBUF - 1) % NBUF],
                               sems[(b + NBUF - 1) % NBUF])

                    pltpu.sync_copy(bufs[b], agg_sh.at[dst_v.at[j + b]],
                                    add=True)
                return carry2

            return lax.fori_loop(0, G // NBUF, quad, carry)

        lax.fori_loop(0, groups, group, 0)
        plsc.subcore_barrier()

        @pl.when(s < NS - 1)
        def _():
            pltpu.sync_copy(agg_sh.at[pl.ds(s * ZR, ZR)],
                            out_hbm.at[pl.ds(c * N + s * ZR, ZR)])

        @pl.when(s == NS - 1)
        def _():
            pltpu.sync_copy(agg_sh.at[pl.ds((NS - 1) * ZR, TAIL)],
                            out_hbm.at[pl.ds(c * N + (NS - 1) * ZR, TAIL)])

    return pl.kernel(
        body,
        out_type=jax.ShapeDtypeStruct((NC * N, HALF), jnp.float32),
        mesh=plsc.VectorSubcoreMesh(core_axis_name="c", subcore_axis_name="s"),
        scratch_types=[
            pltpu.VMEM((G, K), jnp.int32),
            pltpu.VMEM((G, K), jnp.int32),
            pltpu.VMEM((K, HALF), jnp.float32),
            pltpu.VMEM((K, HALF), jnp.float32),
            pltpu.VMEM((K, HALF), jnp.float32),
            pltpu.VMEM((K, HALF), jnp.float32),
            pltpu.VMEM((K, HALF), jnp.float32),
            pltpu.VMEM((K, HALF), jnp.float32),
            pltpu.VMEM((K, HALF), jnp.float32),
            pltpu.VMEM((K, HALF), jnp.float32),
            pltpu.VMEM_SHARED((AGG_ROWS, HALF), jnp.float32),
            pltpu.SemaphoreType.DMA,
            pltpu.SemaphoreType.DMA,
            pltpu.SemaphoreType.DMA,
            pltpu.SemaphoreType.DMA,
            pltpu.SemaphoreType.DMA,
            pltpu.SemaphoreType.DMA,
            pltpu.SemaphoreType.DMA,
            pltpu.SemaphoreType.DMA,
        ],
    )


CHUNKS_ES = 320
_edge_agg_l1 = _make_edge_agg(CHUNKS_ES, True)
_edge_agg128 = _make_edge_agg(CHUNKS, False)


def _gru_tail(m_agg, gh, h_full, o_ref):
    gi = m_agg
    r = jax.nn.sigmoid(gi[:, :HID] + gh[:, :HID])
    z = jax.nn.sigmoid(gi[:, HID:2 * HID] + gh[:, HID:2 * HID])
    n = jnp.tanh(gi[:, 2 * HID:] + r * gh[:, 2 * HID:])
    h_new = (1.0 - z) * n + z * h_full
    o_ref[0] = h_new[:, :HALF]
    o_ref[1] = h_new[:, HALF:]


def _gru1_body(s_ref, x_ref, w_ref, wi_ref, wh_ref, bi_ref, bh_ref, o_ref):
    s = s_ref[0] + s_ref[1]
    m_agg = jnp.dot(s, w_ref[...], preferred_element_type=jnp.float32)
    gi = jnp.dot(m_agg, wi_ref[...], preferred_element_type=jnp.float32) + bi_ref[...]
    gh = jnp.dot(x_ref[...], wh_ref[...], preferred_element_type=jnp.float32) + bh_ref[...]
    h_full = jnp.concatenate(
        [x_ref[...], jnp.zeros((RB, HALF), jnp.float32)], axis=1)
    _gru_tail(gi, gh, h_full, o_ref)


def _gru1(s2, x, w1, wiT, whT1, bi, bh):
    return pl.pallas_call(
        _gru1_body,
        grid=(GRID,),
        in_specs=[pl.BlockSpec((NC, RB, HALF), lambda i: (0, i, 0)),
                  pl.BlockSpec((RB, HALF), lambda i: (i, 0)),
                  pl.BlockSpec((HALF, HID), lambda i: (0, 0)),
                  pl.BlockSpec((HID, 3 * HID), lambda i: (0, 0)),
                  pl.BlockSpec((HALF, 3 * HID), lambda i: (0, 0)),
                  pl.BlockSpec((1, 3 * HID), lambda i: (0, 0)),
                  pl.BlockSpec((1, 3 * HID), lambda i: (0, 0))],
        out_specs=pl.BlockSpec((NC, RB, HALF), lambda i: (0, i, 0)),
        out_shape=jax.ShapeDtypeStruct((NC, N, HALF), jnp.float32),
    )(s2, x, w1, wiT, whT1, bi, bh)


def _gru23_body(s_ref, h_ref, w_ref, wi_ref, wh_ref, bi_ref, bh_ref, o_ref):
    m_agg = (jnp.dot(s_ref[0], w_ref[:HALF, :], preferred_element_type=jnp.float32)
             + jnp.dot(s_ref[1], w_ref[HALF:, :], preferred_element_type=jnp.float32))
    gi = jnp.dot(m_agg, wi_ref[...], preferred_element_type=jnp.float32) + bi_ref[...]
    gh = (jnp.dot(h_ref[0], wh_ref[:HALF, :], preferred_element_type=jnp.float32)
          + jnp.dot(h_ref[1], wh_ref[HALF:, :], preferred_element_type=jnp.float32)
          + bh_ref[...])
    h_full = jnp.concatenate([h_ref[0], h_ref[1]], axis=1)
    _gru_tail(gi, gh, h_full, o_ref)


def _gru23(s2, h2, w, wiT, whT, bi, bh):
    return pl.pallas_call(
        _gru23_body,
        grid=(GRID,),
        in_specs=[pl.BlockSpec((NC, RB, HALF), lambda i: (0, i, 0)),
                  pl.BlockSpec((NC, RB, HALF), lambda i: (0, i, 0)),
                  pl.BlockSpec((HID, HID), lambda i: (0, 0)),
                  pl.BlockSpec((HID, 3 * HID), lambda i: (0, 0)),
                  pl.BlockSpec((HID, 3 * HID), lambda i: (0, 0)),
                  pl.BlockSpec((1, 3 * HID), lambda i: (0, 0)),
                  pl.BlockSpec((1, 3 * HID), lambda i: (0, 0))],
        out_specs=pl.BlockSpec((NC, RB, HALF), lambda i: (0, i, 0)),
        out_shape=jax.ShapeDtypeStruct((NC, N, HALF), jnp.float32),
    )(s2, h2, w, wiT, whT, bi, bh)


def _pool_body(h_ref, b_ref, w_ref, lb_ref, o_ref, sums, cnt):
    i = pl.program_id(0)

    @pl.when(i == 0)
    def _():
        sums[...] = jnp.zeros_like(sums)
        cnt[...] = jnp.zeros_like(cnt)

    gid = lax.broadcasted_iota(jnp.int32, (NGRAPHS, RB), 0)
    oh = (b_ref[0] == gid).astype(jnp.float32)
    hcat = jnp.concatenate([h_ref[0], h_ref[1]], axis=1)
    sums[...] += jnp.dot(oh, hcat, preferred_element_type=jnp.float32)
    cnt[...] += jnp.broadcast_to(jnp.sum(oh, axis=1, keepdims=True),
                                 (NGRAPHS, HALF))

    @pl.when(i == GRID - 1)
    def _():
        pooled = sums[...] / jnp.maximum(cnt[:, 0:1], 1.0)
        o_ref[...] = (jnp.dot(pooled, w_ref[...],
                              preferred_element_type=jnp.float32)
                      + lb_ref[...])


def _pool(h2, batch2, lin_WT, lin_b2):
    return pl.pallas_call(
        _pool_body,
        grid=(GRID,),
        in_specs=[pl.BlockSpec((NC, RB, HALF), lambda i: (0, i, 0)),
                  pl.BlockSpec((1, 1, RB), lambda i: (i, 0, 0)),
                  pl.BlockSpec((HID, NCLS), lambda i: (0, 0)),
                  pl.BlockSpec((1, NCLS), lambda i: (0, 0))],
        out_specs=pl.BlockSpec((NGRAPHS, NCLS), lambda i: (0, 0)),
        out_shape=jax.ShapeDtypeStruct((NGRAPHS, NCLS), jnp.float32),
        scratch_shapes=[pltpu.VMEM((NGRAPHS, HID), jnp.float32),
                        pltpu.VMEM((NGRAPHS, HALF), jnp.float32)],
    )(h2, batch2, lin_WT, lin_b2)


def kernel(x, edge_index, batch, weight, W_ih, W_hh, b_ih, b_hh, lin_W, lin_b):
    src = edge_index[0].astype(jnp.int32)
    dst = edge_index[1].astype(jnp.int32)
    batch = batch.astype(jnp.int32)

    pad = E_PAD - E
    srcp = jnp.concatenate([src, jnp.zeros((pad,), jnp.int32)])
    trash = N + jnp.arange(pad, dtype=jnp.int32) % (AGG_ROWS - N)
    dstp = jnp.concatenate([dst, trash])
    src_st = jnp.stack([srcp, srcp + N]).reshape(NC, NS, CHUNKS, K)
    dst3 = dstp.reshape(NS, CHUNKS, K)
    src_es = srcp.reshape(NC, NS, CHUNKS_ES, K)
    dst_es = dstp.reshape(NC, NS, CHUNKS_ES, K)
    zeros128 = jnp.zeros((AGG_ROWS, HALF), jnp.float32)

    wiT = W_ih.T
    whT = W_hh.T
    bi = b_ih.reshape(1, 3 * HID)
    bh = b_hh.reshape(1, 3 * HID)

    s1 = _edge_agg_l1(x, src_es, dst_es, zeros128)
    h2 = _gru1(s1.reshape(NC, N, HALF), x, weight[0][:HALF],
               wiT, whT[:HALF], bi, bh)

    for i in range(1, NLAYERS):
        s = _edge_agg128(h2.reshape(NC * N, HALF), src_st, dst3, zeros128)
        h2 = _gru23(s.reshape(NC, N, HALF), h2, weight[i], wiT, whT, bi, bh)

    batch2 = batch.reshape(GRID, 1, RB)
    return _pool(h2, batch2, lin_W.T, lin_b.reshape(1, NCLS))
